# baseline (device time: 100130 ns/iter reference)
import jax
import jax.numpy as jnp
from jax import lax
from jax.experimental import pallas as pl
from jax.experimental.pallas import tpu as pltpu

N_DEV = 16
B, SQ, D_MODEL = 2, 128, 512
HQ, DH = 4, 64
SKV_LOC = 128
BLK = 64
ROWS = B * SQ
COLS = HQ * DH


def kernel(x, Wq, K_ext, V_ext, Wo):
    def body(x_ref, wq_ref, k_ref, v_ref, wo_ref, out_ref,
             acc_o, acc_s, comm_o, comm_s,
             send_o, recv_o, send_s, recv_s):
        my = lax.axis_index("i")
        left = (my + N_DEV - 1) % N_DEV
        right = (my + 1) % N_DEV

        row_blk = lax.broadcasted_iota(jnp.int32, (SQ, SKV_LOC), 0) // BLK
        col_blk = lax.broadcasted_iota(jnp.int32, (SQ, SKV_LOC), 1) // BLK + 2 * my
        mask = (row_blk == col_blk) | (col_blk == 0) | ((row_blk + col_blk) % 3 == 0)

        for b in range(B):
            q_b = jnp.dot(x_ref[b], wq_ref[...],
                          preferred_element_type=jnp.float32)
            for h in range(HQ):
                q_bh = q_b[:, h * DH:(h + 1) * DH]
                k_bh = k_ref[b, :, h, :]
                v_bh = v_ref[b, :, h, :]
                scores = lax.dot_general(
                    q_bh, k_bh, (((1,), (1,)), ((), ())),
                    preferred_element_type=jnp.float32) * 0.125
                w = jnp.where(mask, jnp.exp(scores), 0.0)
                o_bh = jnp.dot(w, v_bh, preferred_element_type=jnp.float32)
                s_bh = jnp.sum(w, axis=1, keepdims=True)
                acc_o[pl.ds(b * SQ, SQ), pl.ds(h * DH, DH)] = o_bh
                comm_o[0, pl.ds(b * SQ, SQ), pl.ds(h * DH, DH)] = o_bh
                acc_s[pl.ds(b * SQ, SQ), pl.ds(h, 1)] = s_bh
                comm_s[0, pl.ds(b * SQ, SQ), pl.ds(h, 1)] = s_bh
        acc_s[:, pl.ds(HQ, 4)] = jnp.zeros((ROWS, 4), jnp.float32)
        comm_s[0, :, pl.ds(HQ, 4)] = jnp.zeros((ROWS, 4), jnp.float32)

        barrier = pltpu.get_barrier_semaphore()
        for nbr in (left, right):
            pl.semaphore_signal(barrier, inc=1, device_id=(nbr,),
                                device_id_type=pl.DeviceIdType.MESH)
        pl.semaphore_wait(barrier, 2)

        for hop in range(N_DEV - 1):
            snd, rcv = hop % 2, (hop + 1) % 2
            ro = pltpu.make_async_remote_copy(
                src_ref=comm_o.at[snd], dst_ref=comm_o.at[rcv],
                send_sem=send_o.at[snd], recv_sem=recv_o.at[rcv],
                device_id=(right,), device_id_type=pl.DeviceIdType.MESH)
            rs = pltpu.make_async_remote_copy(
                src_ref=comm_s.at[snd], dst_ref=comm_s.at[rcv],
                send_sem=send_s.at[snd], recv_sem=recv_s.at[rcv],
                device_id=(right,), device_id_type=pl.DeviceIdType.MESH)
            ro.start()
            rs.start()
            ro.wait()
            rs.wait()
            acc_o[...] += comm_o[rcv]
            acc_s[...] += comm_s[rcv]

        for b in range(B):
            out_b = jnp.zeros((SQ, D_MODEL), jnp.float32)
            for h in range(HQ):
                ctx = (acc_o[pl.ds(b * SQ, SQ), pl.ds(h * DH, DH)]
                       / acc_s[pl.ds(b * SQ, SQ), pl.ds(h, 1)])
                out_b = out_b + jnp.dot(ctx, wo_ref[pl.ds(h * DH, DH), :],
                                        preferred_element_type=jnp.float32)
            out_ref[b] = out_b

    return pl.pallas_call(
        body,
        out_shape=jax.ShapeDtypeStruct((B, SQ, D_MODEL), jnp.float32),
        in_specs=[pl.BlockSpec(memory_space=pltpu.VMEM)] * 5,
        out_specs=pl.BlockSpec(memory_space=pltpu.VMEM),
        scratch_shapes=[
            pltpu.VMEM((ROWS, COLS), jnp.float32),
            pltpu.VMEM((ROWS, 8), jnp.float32),
            pltpu.VMEM((2, ROWS, COLS), jnp.float32),
            pltpu.VMEM((2, ROWS, 8), jnp.float32),
            pltpu.SemaphoreType.DMA((2,)),
            pltpu.SemaphoreType.DMA((2,)),
            pltpu.SemaphoreType.DMA((2,)),
            pltpu.SemaphoreType.DMA((2,)),
        ],
        compiler_params=pltpu.CompilerParams(collective_id=0),
    )(x, Wq, K_ext, V_ext, Wo)


# device time: 39194 ns/iter; 2.5547x vs baseline; 2.5547x over previous
import jax
import jax.numpy as jnp
from jax import lax
from jax.experimental import pallas as pl
from jax.experimental.pallas import tpu as pltpu

N_DEV = 16
B, SQ, D_MODEL = 2, 128, 512
HQ, DH = 4, 64
SKV_LOC = 128
BLK = 64
ROWS = B * SQ
COLS = HQ * DH


def kernel(x, Wq, K_ext, V_ext, Wo):
    def body(x_ref, wq_ref, k_ref, v_ref, wo_ref, out_ref,
             acc_o, acc_s, comm_o, comm_s,
             send_o, recv_o, send_s, recv_s):
        my = lax.axis_index("i")

        row_blk = lax.broadcasted_iota(jnp.int32, (SQ, SKV_LOC), 0) // BLK
        col_blk = lax.broadcasted_iota(jnp.int32, (SQ, SKV_LOC), 1) // BLK + 2 * my
        mask = (row_blk == col_blk) | (col_blk == 0) | ((row_blk + col_blk) % 3 == 0)

        for b in range(B):
            q_b = jnp.dot(x_ref[b], wq_ref[...],
                          preferred_element_type=jnp.float32)
            for h in range(HQ):
                q_bh = q_b[:, h * DH:(h + 1) * DH]
                k_bh = k_ref[b, :, h, :]
                v_bh = v_ref[b, :, h, :]
                scores = lax.dot_general(
                    q_bh, k_bh, (((1,), (1,)), ((), ())),
                    preferred_element_type=jnp.float32) * 0.125
                w = jnp.where(mask, jnp.exp(scores), 0.0)
                o_bh = jnp.dot(w, v_bh, preferred_element_type=jnp.float32)
                s_bh = jnp.sum(w, axis=1, keepdims=True)
                acc_o[pl.ds(b * SQ, SQ), pl.ds(h * DH, DH)] = o_bh
                acc_s[pl.ds(b * SQ, SQ), pl.ds(h, 1)] = s_bh
        acc_s[:, pl.ds(HQ, 4)] = jnp.zeros((ROWS, 4), jnp.float32)

        n_rounds = N_DEV.bit_length() - 1
        partners = [my ^ (1 << k) for k in range(n_rounds)]
        barrier = pltpu.get_barrier_semaphore()
        for p in partners:
            pl.semaphore_signal(barrier, inc=1, device_id=(p,),
                                device_id_type=pl.DeviceIdType.MESH)
        pl.semaphore_wait(barrier, n_rounds)

        for k in range(n_rounds):
            ro = pltpu.make_async_remote_copy(
                src_ref=acc_o, dst_ref=comm_o.at[k],
                send_sem=send_o.at[k], recv_sem=recv_o.at[k],
                device_id=(partners[k],), device_id_type=pl.DeviceIdType.MESH)
            rs = pltpu.make_async_remote_copy(
                src_ref=acc_s, dst_ref=comm_s.at[k],
                send_sem=send_s.at[k], recv_sem=recv_s.at[k],
                device_id=(partners[k],), device_id_type=pl.DeviceIdType.MESH)
            ro.start()
            rs.start()
            ro.wait()
            rs.wait()
            acc_o[...] += comm_o[k]
            acc_s[...] += comm_s[k]

        for b in range(B):
            out_b = jnp.zeros((SQ, D_MODEL), jnp.float32)
            for h in range(HQ):
                ctx = (acc_o[pl.ds(b * SQ, SQ), pl.ds(h * DH, DH)]
                       / acc_s[pl.ds(b * SQ, SQ), pl.ds(h, 1)])
                out_b = out_b + jnp.dot(ctx, wo_ref[pl.ds(h * DH, DH), :],
                                        preferred_element_type=jnp.float32)
            out_ref[b] = out_b

    return pl.pallas_call(
        body,
        out_shape=jax.ShapeDtypeStruct((B, SQ, D_MODEL), jnp.float32),
        in_specs=[pl.BlockSpec(memory_space=pltpu.VMEM)] * 5,
        out_specs=pl.BlockSpec(memory_space=pltpu.VMEM),
        scratch_shapes=[
            pltpu.VMEM((ROWS, COLS), jnp.float32),
            pltpu.VMEM((ROWS, 8), jnp.float32),
            pltpu.VMEM((4, ROWS, COLS), jnp.float32),
            pltpu.VMEM((4, ROWS, 8), jnp.float32),
            pltpu.SemaphoreType.DMA((4,)),
            pltpu.SemaphoreType.DMA((4,)),
            pltpu.SemaphoreType.DMA((4,)),
            pltpu.SemaphoreType.DMA((4,)),
        ],
        compiler_params=pltpu.CompilerParams(collective_id=0),
    )(x, Wq, K_ext, V_ext, Wo)


# device time: 32263 ns/iter; 3.1036x vs baseline; 1.2148x over previous
import jax
import jax.numpy as jnp
from jax import lax
from jax.experimental import pallas as pl
from jax.experimental.pallas import tpu as pltpu

N_DEV = 16
N_ROUNDS = 4
B, SQ, D_MODEL = 2, 128, 512
HQ, DH = 4, 64
SKV_LOC = 128
BLK = 64
COLS = HQ * DH


def kernel(x, Wq, K_ext, V_ext, Wo):
    def body(x_ref, wq_ref, k_ref, v_ref, wo_ref, out_ref,
             acc_o, acc_s, comm_o, comm_s,
             send_o, recv_o, send_s, recv_s):
        my = lax.axis_index("i")
        partners = [my ^ (1 << k) for k in range(N_ROUNDS)]

        row_blk = lax.broadcasted_iota(jnp.int32, (SQ, SKV_LOC), 0) // BLK
        col_blk = lax.broadcasted_iota(jnp.int32, (SQ, SKV_LOC), 1) // BLK + 2 * my
        mask = (row_blk == col_blk) | (col_blk == 0) | ((row_blk + col_blk) % 3 == 0)

        def compute_partials(b):
            q_b = jnp.dot(x_ref[b], wq_ref[...],
                          preferred_element_type=jnp.float32)
            for h in range(HQ):
                q_bh = q_b[:, h * DH:(h + 1) * DH]
                k_bh = k_ref[b, :, h, :]
                v_bh = v_ref[b, :, h, :]
                scores = lax.dot_general(
                    q_bh, k_bh, (((1,), (1,)), ((), ())),
                    preferred_element_type=jnp.float32) * 0.125
                w = jnp.where(mask, jnp.exp(scores), 0.0)
                acc_o[b, :, pl.ds(h * DH, DH)] = jnp.dot(
                    w, v_bh, preferred_element_type=jnp.float32)
                acc_s[b, :, pl.ds(h, 1)] = jnp.sum(w, axis=1, keepdims=True)
            acc_s[b, :, pl.ds(HQ, 4)] = jnp.zeros((SQ, 4), jnp.float32)

        def exchange(k, half):
            ro = pltpu.make_async_remote_copy(
                src_ref=acc_o.at[half], dst_ref=comm_o.at[k, half],
                send_sem=send_o.at[k, half], recv_sem=recv_o.at[k, half],
                device_id=(partners[k],), device_id_type=pl.DeviceIdType.MESH)
            rs = pltpu.make_async_remote_copy(
                src_ref=acc_s.at[half], dst_ref=comm_s.at[k, half],
                send_sem=send_s.at[k, half], recv_sem=recv_s.at[k, half],
                device_id=(partners[k],), device_id_type=pl.DeviceIdType.MESH)
            return ro, rs

        def accumulate(k, half):
            acc_o[half] += comm_o[k, half]
            acc_s[half] += comm_s[k, half]

        def finalize(b):
            out_b = jnp.zeros((SQ, D_MODEL), jnp.float32)
            for h in range(HQ):
                ctx = (acc_o[b, :, pl.ds(h * DH, DH)]
                       / acc_s[b, :, pl.ds(h, 1)])
                out_b = out_b + jnp.dot(ctx, wo_ref[pl.ds(h * DH, DH), :],
                                        preferred_element_type=jnp.float32)
            out_ref[b] = out_b

        compute_partials(0)

        barrier = pltpu.get_barrier_semaphore()
        for p in partners:
            pl.semaphore_signal(barrier, inc=1, device_id=(p,),
                                device_id_type=pl.DeviceIdType.MESH)
        pl.semaphore_wait(barrier, N_ROUNDS)

        a_o, a_s = exchange(0, 0)
        a_o.start()
        a_s.start()

        compute_partials(1)
        b_o, b_s = exchange(0, 1)
        b_o.start()
        b_s.start()

        for k in range(N_ROUNDS):
            a_o.wait()
            a_s.wait()
            accumulate(k, 0)
            if k + 1 < N_ROUNDS:
                a_o, a_s = exchange(k + 1, 0)
                a_o.start()
                a_s.start()
            b_o.wait()
            b_s.wait()
            accumulate(k, 1)
            if k + 1 < N_ROUNDS:
                b_o, b_s = exchange(k + 1, 1)
                b_o.start()
                b_s.start()

        finalize(0)
        finalize(1)

    return pl.pallas_call(
        body,
        out_shape=jax.ShapeDtypeStruct((B, SQ, D_MODEL), jnp.float32),
        in_specs=[pl.BlockSpec(memory_space=pltpu.VMEM)] * 5,
        out_specs=pl.BlockSpec(memory_space=pltpu.VMEM),
        scratch_shapes=[
            pltpu.VMEM((B, SQ, COLS), jnp.float32),
            pltpu.VMEM((B, SQ, 8), jnp.float32),
            pltpu.VMEM((N_ROUNDS, B, SQ, COLS), jnp.float32),
            pltpu.VMEM((N_ROUNDS, B, SQ, 8), jnp.float32),
            pltpu.SemaphoreType.DMA((N_ROUNDS, B)),
            pltpu.SemaphoreType.DMA((N_ROUNDS, B)),
            pltpu.SemaphoreType.DMA((N_ROUNDS, B)),
            pltpu.SemaphoreType.DMA((N_ROUNDS, B)),
        ],
        compiler_params=pltpu.CompilerParams(collective_id=0),
    )(x, Wq, K_ext, V_ext, Wo)


# device time: 28000 ns/iter; 3.5761x vs baseline; 1.1522x over previous
import jax
import jax.numpy as jnp
from jax import lax
from jax.experimental import pallas as pl
from jax.experimental.pallas import tpu as pltpu

N_DEV = 16
N_ROUNDS = 4
B, SQ, D_MODEL = 2, 128, 512
HQ, DH = 4, 64
SKV_LOC = 128
BLK = 64
COLS = HQ * DH
PROWS = SQ + 8


def kernel(x, Wq, K_ext, V_ext, Wo):
    def body(x_ref, wq_ref, k_ref, v_ref, wo_ref, out_ref,
             acc, comm, send_sems, recv_sems):
        my = lax.axis_index("i")
        partners = [my ^ (1 << k) for k in range(N_ROUNDS)]

        row_blk = lax.broadcasted_iota(jnp.int32, (SQ, SKV_LOC), 0) // BLK
        col_blk = lax.broadcasted_iota(jnp.int32, (SQ, SKV_LOC), 1) // BLK + 2 * my
        mask = (row_blk == col_blk) | (col_blk == 0) | ((row_blk + col_blk) % 3 == 0)
        ones_row = jnp.ones((1, SKV_LOC), jnp.float32)

        def compute_partials(b):
            acc[b, pl.ds(SQ, 8), :] = jnp.zeros((8, COLS), jnp.float32)
            q_b = jnp.dot(x_ref[b], wq_ref[...],
                          preferred_element_type=jnp.float32)
            for h in range(HQ):
                q_bh = q_b[:, h * DH:(h + 1) * DH]
                k_bh = k_ref[b, :, h, :]
                v_bh = v_ref[b, :, h, :]
                scores = lax.dot_general(
                    q_bh, k_bh, (((1,), (1,)), ((), ())),
                    preferred_element_type=jnp.float32) * 0.125
                w = jnp.where(mask, jnp.exp(scores), 0.0)
                acc[b, pl.ds(0, SQ), pl.ds(h * DH, DH)] = jnp.dot(
                    w, v_bh, preferred_element_type=jnp.float32)
                acc[b, pl.ds(SQ + h, 1), pl.ds(0, SQ)] = lax.dot_general(
                    ones_row, w, (((1,), (1,)), ((), ())),
                    preferred_element_type=jnp.float32)

        def exchange(k, half):
            r = pltpu.make_async_remote_copy(
                src_ref=acc.at[half], dst_ref=comm.at[k, half],
                send_sem=send_sems.at[k, half], recv_sem=recv_sems.at[k, half],
                device_id=(partners[k],), device_id_type=pl.DeviceIdType.MESH)
            r.start()
            return r

        def finalize(b):
            s_cols = jnp.transpose(
                acc[b, pl.ds(SQ, 8), pl.ds(0, SQ)])
            out_b = jnp.zeros((SQ, D_MODEL), jnp.float32)
            for h in range(HQ):
                ctx = acc[b, pl.ds(0, SQ), pl.ds(h * DH, DH)] / s_cols[:, h:h + 1]
                out_b = out_b + jnp.dot(ctx, wo_ref[pl.ds(h * DH, DH), :],
                                        preferred_element_type=jnp.float32)
            out_ref[b] = out_b

        compute_partials(0)

        barrier = pltpu.get_barrier_semaphore()
        for p in partners:
            pl.semaphore_signal(barrier, inc=1, device_id=(p,),
                                device_id_type=pl.DeviceIdType.MESH)
        pl.semaphore_wait(barrier, N_ROUNDS)

        rdma_a = exchange(0, 0)
        compute_partials(1)
        rdma_b = exchange(0, 1)

        for k in range(N_ROUNDS):
            rdma_a.wait()
            acc[0] += comm[k, 0]
            if k + 1 < N_ROUNDS:
                rdma_a = exchange(k + 1, 0)
            rdma_b.wait()
            acc[1] += comm[k, 1]
            if k + 1 < N_ROUNDS:
                rdma_b = exchange(k + 1, 1)

        finalize(0)
        finalize(1)

    return pl.pallas_call(
        body,
        out_shape=jax.ShapeDtypeStruct((B, SQ, D_MODEL), jnp.float32),
        in_specs=[pl.BlockSpec(memory_space=pltpu.VMEM)] * 5,
        out_specs=pl.BlockSpec(memory_space=pltpu.VMEM),
        scratch_shapes=[
            pltpu.VMEM((B, PROWS, COLS), jnp.float32),
            pltpu.VMEM((N_ROUNDS, B, PROWS, COLS), jnp.float32),
            pltpu.SemaphoreType.DMA((N_ROUNDS, B)),
            pltpu.SemaphoreType.DMA((N_ROUNDS, B)),
        ],
        compiler_params=pltpu.CompilerParams(collective_id=0),
    )(x, Wq, K_ext, V_ext, Wo)
